# baseline (speedup 1.0000x reference)
import functools

import numpy as np
import jax
import jax.numpy as jnp
from jax import lax
from jax.experimental import pallas as pl
from jax.experimental.pallas import tpu as pltpu
from jax.experimental.pallas import tpu_sc as plsc

ALPHA = 0.1
THETA = 0.5

_NC = 2
_NS = 16


@functools.cache
def _make_spmm(N, E, H):
    K = 80
    EP = E // (_NC * _NS)
    NCH = EP // K
    assert EP * _NC * _NS == E and NCH * K == EP
    RP = (N // _NS) // 8 * 8
    TAIL = N - RP * _NS
    assert RP % K == 0 or (RP // K) * K + 64 == RP
    assert 0 <= TAIL <= K and TAIL % 8 == 0 and H % 16 == 0
    mesh = plsc.VectorSubcoreMesh(core_axis_name="c", subcore_axis_name="s")

    NBUF = 4
    NI = 2 * NBUF

    _DN = lax.GatherDimensionNumbers(
        offset_dims=(), collapsed_slice_dims=(0,), start_index_map=(0,))

    def body(x_hbm, e_hbm, out_hbm, iring, bufs, agg_sh,
             gsems, ssems, isems):
        c = lax.axis_index("c")
        s = lax.axis_index("s")
        wid = c * _NS + s

        def buf_rows(b):
            return bufs.at[pl.ds(pl.multiple_of(b * K, 8), K)]

        z16 = jnp.zeros((16,), jnp.float32)

        def zrow(i, carry):
            for f in range(H // 16):
                bufs[i, pl.ds(f * 16, 16)] = z16
            return carry

        lax.fori_loop(0, K, zrow, 0)
        nfull = RP // K
        for r in range(nfull):
            pltpu.sync_copy(buf_rows(0), agg_sh.at[pl.ds(s * RP + r * K, K)])
        rem = RP - nfull * K
        if rem:
            pltpu.sync_copy(bufs.at[pl.ds(0, rem)],
                            agg_sh.at[pl.ds(s * RP + nfull * K, rem)])
        if TAIL:
            @pl.when(jnp.logical_and(s == 0, c >= 0))
            def _zero_tail():
                pltpu.sync_copy(bufs.at[pl.ds(0, TAIL)],
                                agg_sh.at[pl.ds(RP * _NS, TAIL)])
        plsc.subcore_barrier()

        def stage_idx(k, i):
            return pltpu.async_copy(
                e_hbm.at[wid, k],
                iring.at[pl.ds(pl.multiple_of(i * 3, 1), 3)], isems.at[i])

        def gather(i, b):
            return pltpu.async_copy(
                x_hbm.at[pl.ds(pl.multiple_of(0, 8), K)], buf_rows(b),
                gsems.at[b])

        for k0 in range(min(NI, NCH)):
            stage_idx(k0, k0)
        for k0 in range(min(NBUF, NCH)):
            pltpu.make_async_copy(
                e_hbm.at[wid, k0], iring.at[pl.ds(k0 * 3, 3)],
                isems.at[k0]).wait()
            gather(k0, k0)

        def step(k, carry):
            b = lax.rem(k, NBUF)
            i = lax.rem(k, NI)
            pltpu.make_async_copy(
                x_hbm.at[pl.ds(pl.multiple_of(0, 8), K)], buf_rows(b),
                gsems.at[b]).wait()
            brow = b * K

            def scale(g, c2):
                wv = plsc.bitcast(iring[i * 3 + 2, pl.ds(g * 16, 16)],
                                  jnp.float32)
                for t in range(16):
                    wb = lax.gather(
                        wv, jnp.full((16, 1), t, jnp.int32), _DN,
                        slice_sizes=(1,),
                        mode=lax.GatherScatterMode.PROMISE_IN_BOUNDS)
                    j = brow + g * 16 + t
                    for f in range(H // 16):
                        sl = pl.ds(f * 16, 16)
                        bufs[j, sl] = bufs[j, sl] * wb
                return c2

            pltpu.async_copy(buf_rows(b), agg_sh.at[iring.at[i * 3 + 1]],
                             ssems.at[b], add=True)

            @pl.when(jnp.logical_and(k >= 1, k - 1 + NBUF < NCH))
            def _refill():
                kp = k - 1
                bp = lax.rem(kp, NBUF)
                iold = lax.rem(kp, NI)
                ip = lax.rem(kp + NBUF, NI)
                pltpu.make_async_copy(buf_rows(bp),
                                      agg_sh.at[iring.at[iold * 3 + 1]],
                                      ssems.at[bp]).wait()

                @pl.when(kp + NI < NCH)
                def _stage_ahead():
                    stage_idx(kp + NI, iold)

                pltpu.make_async_copy(
                    e_hbm.at[wid, kp + NBUF],
                    iring.at[pl.ds(ip * 3, 3)], isems.at[ip]).wait()
                gather(ip, bp)
            return carry

        lax.fori_loop(0, NCH, step, 0)
        for d in range(NBUF):
            kd = NCH - NBUF + d
            pltpu.make_async_copy(
                buf_rows(kd % NBUF),
                agg_sh.at[iring.at[(kd % NI) * 3 + 1]],
                ssems.at[kd % NBUF]).wait()
        plsc.subcore_barrier()
        dst_row = pl.multiple_of(c * N + s * RP, 8)
        pltpu.sync_copy(agg_sh.at[pl.ds(s * RP, RP)],
                        out_hbm.at[pl.ds(dst_row, RP)])
        if TAIL:
            @pl.when(jnp.logical_and(s == 0, c >= 0))
            def _write_tail():
                tr = pl.multiple_of(c * N + RP * _NS, 8)
                pltpu.sync_copy(agg_sh.at[pl.ds(RP * _NS, TAIL)],
                                out_hbm.at[pl.ds(tr, TAIL)])

    k_fn = pl.kernel(
        body,
        out_type=jax.ShapeDtypeStruct((_NC * N, H), jnp.float32),
        mesh=mesh,
        scratch_types=[
            pltpu.VMEM((NI * 3, K), jnp.int32),
            pltpu.VMEM((NBUF * K, H), jnp.float32),
            pltpu.VMEM_SHARED((N, H), jnp.float32),
            pltpu.SemaphoreType.DMA((NBUF,)),
            pltpu.SemaphoreType.DMA((NBUF,)),
            pltpu.SemaphoreType.DMA((NI,)),
        ],
        compiler_params=pltpu.CompilerParams(needs_layout_passes=False),
    )

    NW = _NC * _NS

    def spmm(x, epack):
        return k_fn(x, epack)

    def pack_edges(src, dst, w):
        wbits = lax.bitcast_convert_type(w, jnp.int32)
        e = jnp.stack([src, dst, wbits])
        e = e.reshape(3, NW, NCH, K)
        return e.transpose(1, 2, 0, 3)

    return spmm, pack_edges


def _relu_linear(x, W, b):
    N, F = x.shape
    H = W.shape[1]
    R = 1000

    def body(x_ref, w_ref, b_ref, o_ref):
        o_ref[...] = jnp.maximum(
            jnp.dot(x_ref[...], w_ref[...],
                    preferred_element_type=jnp.float32) + b_ref[...], 0.0)

    return pl.pallas_call(
        body,
        grid=(N // R,),
        in_specs=[pl.BlockSpec((R, F), lambda i: (i, 0)),
                  pl.BlockSpec((F, H), lambda i: (0, 0)),
                  pl.BlockSpec((1, H), lambda i: (0, 0))],
        out_specs=pl.BlockSpec((R, H), lambda i: (i, 0)),
        out_shape=jax.ShapeDtypeStruct((N, H), jnp.float32),
    )(x, W, b)


def _linear(x, W, b):
    N, H = x.shape
    C = W.shape[1]
    R = 1000

    def body(x_ref, w_ref, b_ref, o_ref):
        o_ref[...] = jnp.dot(x_ref[...], w_ref[...],
                             preferred_element_type=jnp.float32) + b_ref[...]

    return pl.pallas_call(
        body,
        grid=(N // R,),
        in_specs=[pl.BlockSpec((R, H), lambda i: (i, 0)),
                  pl.BlockSpec((H, C), lambda i: (0, 0)),
                  pl.BlockSpec((1, C), lambda i: (0, 0))],
        out_specs=pl.BlockSpec((R, C), lambda i: (i, 0)),
        out_shape=jax.ShapeDtypeStruct((N, C), jnp.float32),
    )(x, W, b)


def _gcn2_layer(agg2, x0, x, Wl, beta):
    N, H = x.shape
    R = 1000
    nb = N // R

    def body(a0_ref, a1_ref, x0_ref, x_ref, w_ref, o_ref):
        h = ((1.0 - ALPHA) * (a0_ref[...] + a1_ref[...])
             + ALPHA * x0_ref[...])
        o_ref[...] = jnp.maximum(
            (1.0 - beta) * h
            + beta * jnp.dot(h, w_ref[...], preferred_element_type=jnp.float32)
            + x_ref[...], 0.0)

    return pl.pallas_call(
        body,
        grid=(nb,),
        in_specs=[pl.BlockSpec((R, H), lambda i: (i, 0)),
                  pl.BlockSpec((R, H), lambda i, _nb=nb: (i + _nb, 0)),
                  pl.BlockSpec((R, H), lambda i: (i, 0)),
                  pl.BlockSpec((R, H), lambda i: (i, 0)),
                  pl.BlockSpec((H, H), lambda i: (0, 0))],
        out_specs=pl.BlockSpec((R, H), lambda i: (i, 0)),
        out_shape=jax.ShapeDtypeStruct((N, H), jnp.float32),
    )(agg2, agg2, x0, x, Wl)


def kernel(x, edge_index, edge_weight, W_in, b_in, convW, W_out, b_out):
    N = x.shape[0]
    H = W_in.shape[1]
    E = edge_weight.shape[0]
    L = convW.shape[0]
    src = edge_index[0]
    dst = edge_index[1]
    spmm, pack_edges = _make_spmm(N, E, H)
    epack = pack_edges(src, dst, edge_weight)

    h = _relu_linear(x, W_in, b_in.reshape(1, -1))
    x0 = h
    for l in range(L):
        agg2 = spmm(h, epack)
        beta = float(np.log(THETA / (l + 1) + 1.0))
        h = _gcn2_layer(agg2, x0, h, convW[l], beta)
    return _linear(h, W_out, b_out.reshape(1, -1))


# speedup vs baseline: 2.4870x; 2.4870x over previous
import functools

import numpy as np
import jax
import jax.numpy as jnp
from jax import lax
from jax.experimental import pallas as pl
from jax.experimental.pallas import tpu as pltpu
from jax.experimental.pallas import tpu_sc as plsc

ALPHA = 0.1
THETA = 0.5

_NC = 2
_NS = 16


@functools.cache
def _make_spmm(N, E, H):
    K = 80
    EP = E // (_NC * _NS)
    NCH = EP // K
    assert EP * _NC * _NS == E and NCH * K == EP
    RP = (N // _NS) // 8 * 8
    TAIL = N - RP * _NS
    assert RP % K == 0 or (RP // K) * K + 64 == RP
    assert 0 <= TAIL <= K and TAIL % 8 == 0 and H % 16 == 0
    mesh = plsc.VectorSubcoreMesh(core_axis_name="c", subcore_axis_name="s")

    NBUF = 4
    NI = 2 * NBUF

    _DN = lax.GatherDimensionNumbers(
        offset_dims=(), collapsed_slice_dims=(0,), start_index_map=(0,))

    def body(x_hbm, e_hbm, out_hbm, iring, bufs, agg_sh,
             gsems, ssems, isems):
        c = lax.axis_index("c")
        s = lax.axis_index("s")
        wid = c * _NS + s

        def buf_rows(b):
            return bufs.at[pl.ds(pl.multiple_of(b * K, 8), K)]

        def stage_idx(k, i):
            return pltpu.async_copy(
                e_hbm.at[wid, k],
                iring.at[pl.ds(pl.multiple_of(i * 3, 1), 3)], isems.at[i])

        def gather(i, b):
            return pltpu.async_copy(x_hbm.at[iring.at[i * 3]], buf_rows(b),
                                    gsems.at[b])

        for k0 in range(min(NI, NCH)):
            stage_idx(k0, k0)

        z16 = jnp.zeros((16,), jnp.float32)
        ZR = NBUF * K

        def zrow(i, carry):
            for f in range(H // 16):
                bufs[i, pl.ds(f * 16, 16)] = z16
            return carry

        lax.fori_loop(0, ZR, zrow, 0)
        off = 0
        while off < RP:
            step_r = min(ZR, RP - off)
            pltpu.sync_copy(bufs.at[pl.ds(0, step_r)],
                            agg_sh.at[pl.ds(s * RP + off, step_r)])
            off += step_r
        if TAIL:
            @pl.when(jnp.logical_and(s == 0, c >= 0))
            def _zero_tail():
                pltpu.sync_copy(bufs.at[pl.ds(0, TAIL)],
                                agg_sh.at[pl.ds(RP * _NS, TAIL)])

        for k0 in range(min(NBUF, NCH)):
            pltpu.make_async_copy(
                e_hbm.at[wid, k0], iring.at[pl.ds(k0 * 3, 3)],
                isems.at[k0]).wait()
            gather(k0, k0)
        plsc.subcore_barrier()

        def step(k, carry):
            b = lax.rem(k, NBUF)
            i = lax.rem(k, NI)
            pltpu.make_async_copy(x_hbm.at[iring.at[i * 3]], buf_rows(b),
                                  gsems.at[b]).wait()
            brow = b * K

            def scale(g, c2):
                wv = plsc.bitcast(iring[i * 3 + 2, pl.ds(g * 16, 16)],
                                  jnp.float32)
                for t in range(16):
                    ws = wv[t]
                    j = brow + g * 16 + t
                    for f in range(H // 16):
                        sl = pl.ds(f * 16, 16)
                        bufs[j, sl] = bufs[j, sl] * ws
                return c2

            lax.fori_loop(0, K // 16, scale, 0)
            pltpu.async_copy(buf_rows(b), agg_sh.at[iring.at[i * 3 + 1]],
                             ssems.at[b], add=True)

            @pl.when(jnp.logical_and(k >= 1, k - 1 + NBUF < NCH))
            def _refill():
                kp = k - 1
                bp = lax.rem(kp, NBUF)
                iold = lax.rem(kp, NI)
                ip = lax.rem(kp + NBUF, NI)
                pltpu.make_async_copy(buf_rows(bp),
                                      agg_sh.at[iring.at[iold * 3 + 1]],
                                      ssems.at[bp]).wait()

                @pl.when(kp + NI < NCH)
                def _stage_ahead():
                    stage_idx(kp + NI, iold)

                pltpu.make_async_copy(
                    e_hbm.at[wid, kp + NBUF],
                    iring.at[pl.ds(ip * 3, 3)], isems.at[ip]).wait()
                gather(ip, bp)
            return carry

        lax.fori_loop(0, NCH, step, 0)
        for d in range(NBUF):
            kd = NCH - NBUF + d
            pltpu.make_async_copy(
                buf_rows(kd % NBUF),
                agg_sh.at[iring.at[(kd % NI) * 3 + 1]],
                ssems.at[kd % NBUF]).wait()
        plsc.subcore_barrier()
        dst_row = pl.multiple_of(c * N + s * RP, 8)
        pltpu.sync_copy(agg_sh.at[pl.ds(s * RP, RP)],
                        out_hbm.at[pl.ds(dst_row, RP)])
        if TAIL:
            @pl.when(jnp.logical_and(s == 0, c >= 0))
            def _write_tail():
                tr = pl.multiple_of(c * N + RP * _NS, 8)
                pltpu.sync_copy(agg_sh.at[pl.ds(RP * _NS, TAIL)],
                                out_hbm.at[pl.ds(tr, TAIL)])

    k_fn = pl.kernel(
        body,
        out_type=jax.ShapeDtypeStruct((_NC * N, H), jnp.float32),
        mesh=mesh,
        scratch_types=[
            pltpu.VMEM((NI * 3, K), jnp.int32),
            pltpu.VMEM((NBUF * K, H), jnp.float32),
            pltpu.VMEM_SHARED((N, H), jnp.float32),
            pltpu.SemaphoreType.DMA((NBUF,)),
            pltpu.SemaphoreType.DMA((NBUF,)),
            pltpu.SemaphoreType.DMA((NI,)),
        ],
        compiler_params=pltpu.CompilerParams(needs_layout_passes=False),
    )

    NW = _NC * _NS

    def spmm(x, epack):
        return k_fn(x, epack)

    def pack_edges(src, dst, w):
        wbits = lax.bitcast_convert_type(w, jnp.int32)
        e = jnp.stack([src, dst, wbits])
        e = e.reshape(3, NW, NCH, K)
        return e.transpose(1, 2, 0, 3)

    return spmm, pack_edges


def _relu_linear(x, W, b):
    N, F = x.shape
    H = W.shape[1]
    R = 1000

    def body(x_ref, w_ref, b_ref, o_ref):
        o_ref[...] = jnp.maximum(
            jnp.dot(x_ref[...], w_ref[...],
                    preferred_element_type=jnp.float32) + b_ref[...], 0.0)

    return pl.pallas_call(
        body,
        grid=(N // R,),
        in_specs=[pl.BlockSpec((R, F), lambda i: (i, 0)),
                  pl.BlockSpec((F, H), lambda i: (0, 0)),
                  pl.BlockSpec((1, H), lambda i: (0, 0))],
        out_specs=pl.BlockSpec((R, H), lambda i: (i, 0)),
        out_shape=jax.ShapeDtypeStruct((N, H), jnp.float32),
    )(x, W, b)


def _linear(x, W, b):
    N, H = x.shape
    C = W.shape[1]
    R = 1000

    def body(x_ref, w_ref, b_ref, o_ref):
        o_ref[...] = jnp.dot(x_ref[...], w_ref[...],
                             preferred_element_type=jnp.float32) + b_ref[...]

    return pl.pallas_call(
        body,
        grid=(N // R,),
        in_specs=[pl.BlockSpec((R, H), lambda i: (i, 0)),
                  pl.BlockSpec((H, C), lambda i: (0, 0)),
                  pl.BlockSpec((1, C), lambda i: (0, 0))],
        out_specs=pl.BlockSpec((R, C), lambda i: (i, 0)),
        out_shape=jax.ShapeDtypeStruct((N, C), jnp.float32),
    )(x, W, b)


def _gcn2_layer(agg2, x0, x, Wl, beta):
    N, H = x.shape
    R = 1000
    nb = N // R

    def body(a0_ref, a1_ref, x0_ref, x_ref, w_ref, o_ref):
        h = ((1.0 - ALPHA) * (a0_ref[...] + a1_ref[...])
             + ALPHA * x0_ref[...])
        o_ref[...] = jnp.maximum(
            (1.0 - beta) * h
            + beta * jnp.dot(h, w_ref[...], preferred_element_type=jnp.float32)
            + x_ref[...], 0.0)

    return pl.pallas_call(
        body,
        grid=(nb,),
        in_specs=[pl.BlockSpec((R, H), lambda i: (i, 0)),
                  pl.BlockSpec((R, H), lambda i, _nb=nb: (i + _nb, 0)),
                  pl.BlockSpec((R, H), lambda i: (i, 0)),
                  pl.BlockSpec((R, H), lambda i: (i, 0)),
                  pl.BlockSpec((H, H), lambda i: (0, 0))],
        out_specs=pl.BlockSpec((R, H), lambda i: (i, 0)),
        out_shape=jax.ShapeDtypeStruct((N, H), jnp.float32),
    )(agg2, agg2, x0, x, Wl)


def kernel(x, edge_index, edge_weight, W_in, b_in, convW, W_out, b_out):
    N = x.shape[0]
    H = W_in.shape[1]
    E = edge_weight.shape[0]
    L = convW.shape[0]
    src = edge_index[0]
    dst = edge_index[1]
    spmm, pack_edges = _make_spmm(N, E, H)
    epack = pack_edges(src, dst, edge_weight)

    h = _relu_linear(x, W_in, b_in.reshape(1, -1))
    x0 = h
    for l in range(L):
        agg2 = spmm(h, epack)
        beta = float(np.log(THETA / (l + 1) + 1.0))
        h = _gcn2_layer(agg2, x0, h, convW[l], beta)
    return _linear(h, W_out, b_out.reshape(1, -1))


# speedup vs baseline: 2.4887x; 1.0007x over previous
import functools

import numpy as np
import jax
import jax.numpy as jnp
from jax import lax
from jax.experimental import pallas as pl
from jax.experimental.pallas import tpu as pltpu
from jax.experimental.pallas import tpu_sc as plsc

ALPHA = 0.1
THETA = 0.5

_NC = 2
_NS = 16


@functools.cache
def _make_spmm(N, E, H):
    K = 80
    EP = E // (_NC * _NS)
    NCH = EP // K
    assert EP * _NC * _NS == E and NCH * K == EP
    RP = (N // _NS) // 8 * 8
    TAIL = N - RP * _NS
    assert RP % K == 0 or (RP // K) * K + 64 == RP
    assert 0 <= TAIL <= K and TAIL % 8 == 0 and H % 16 == 0
    mesh = plsc.VectorSubcoreMesh(core_axis_name="c", subcore_axis_name="s")

    NBUF = 4
    NI = 2 * NBUF

    def body(x_hbm, e_hbm, out_hbm, iring, bufs, agg_sh,
             gsems, ssems, isems):
        c = lax.axis_index("c")
        s = lax.axis_index("s")
        wid = c * _NS + s

        def buf_rows(b):
            return bufs.at[pl.ds(pl.multiple_of(b * K, 8), K)]

        def stage_idx(k, i):
            return pltpu.async_copy(
                e_hbm.at[wid, k],
                iring.at[pl.ds(pl.multiple_of(i * 3, 1), 3)], isems.at[i])

        def gather(i, b):
            return pltpu.async_copy(x_hbm.at[iring.at[i * 3]], buf_rows(b),
                                    gsems.at[b])

        for k0 in range(min(NI, NCH)):
            stage_idx(k0, k0)

        z16 = jnp.zeros((16,), jnp.float32)
        ZR = NBUF * K

        def zrow(i, carry):
            for f in range(H // 16):
                bufs[i, pl.ds(f * 16, 16)] = z16
            return carry

        lax.fori_loop(0, ZR, zrow, 0)
        off = 0
        while off < RP:
            step_r = min(ZR, RP - off)
            pltpu.sync_copy(bufs.at[pl.ds(0, step_r)],
                            agg_sh.at[pl.ds(s * RP + off, step_r)])
            off += step_r
        if TAIL:
            @pl.when(jnp.logical_and(s == 0, c >= 0))
            def _zero_tail():
                pltpu.sync_copy(bufs.at[pl.ds(0, TAIL)],
                                agg_sh.at[pl.ds(RP * _NS, TAIL)])

        for k0 in range(min(NBUF, NCH)):
            pltpu.make_async_copy(
                e_hbm.at[wid, k0], iring.at[pl.ds(k0 * 3, 3)],
                isems.at[k0]).wait()
            gather(k0, k0)
        plsc.subcore_barrier()

        def step(k, carry):
            b = lax.rem(k, NBUF)
            i = lax.rem(k, NI)
            pltpu.make_async_copy(x_hbm.at[iring.at[i * 3]], buf_rows(b),
                                  gsems.at[b]).wait()
            brow = b * K

            def scale(g, c2):
                wv = plsc.bitcast(iring[i * 3 + 2, pl.ds(g * 16, 16)],
                                  jnp.float32)
                for t in range(16):
                    ws = wv[t]
                    j = brow + g * 16 + t
                    for f in range(H // 16):
                        sl = pl.ds(f * 16, 16)
                        bufs[j, sl] = bufs[j, sl] * ws
                return c2

            lax.fori_loop(0, K // 16, scale, 0)
            pltpu.async_copy(buf_rows(b), agg_sh.at[iring.at[i * 3 + 1]],
                             ssems.at[b], add=True)

            @pl.when(jnp.logical_and(k >= 1, k - 1 + NBUF < NCH))
            def _refill():
                kp = k - 1
                bp = lax.rem(kp, NBUF)
                iold = lax.rem(kp, NI)
                ip = lax.rem(kp + NBUF, NI)
                pltpu.make_async_copy(buf_rows(bp),
                                      agg_sh.at[iring.at[iold * 3 + 1]],
                                      ssems.at[bp]).wait()

                @pl.when(kp + NI < NCH)
                def _stage_ahead():
                    stage_idx(kp + NI, iold)

                pltpu.make_async_copy(
                    e_hbm.at[wid, kp + NBUF],
                    iring.at[pl.ds(ip * 3, 3)], isems.at[ip]).wait()
                gather(ip, bp)
            return carry

        lax.fori_loop(0, NCH, step, 0)
        for d in range(NBUF):
            kd = NCH - NBUF + d
            pltpu.make_async_copy(
                buf_rows(kd % NBUF),
                agg_sh.at[iring.at[(kd % NI) * 3 + 1]],
                ssems.at[kd % NBUF]).wait()
        plsc.subcore_barrier()
        dst_row = pl.multiple_of(c * N + s * RP, 8)
        pltpu.sync_copy(agg_sh.at[pl.ds(s * RP, RP)],
                        out_hbm.at[pl.ds(dst_row, RP)])
        if TAIL:
            @pl.when(jnp.logical_and(s == 0, c >= 0))
            def _write_tail():
                tr = pl.multiple_of(c * N + RP * _NS, 8)
                pltpu.sync_copy(agg_sh.at[pl.ds(RP * _NS, TAIL)],
                                out_hbm.at[pl.ds(tr, TAIL)])

    k_fn = pl.kernel(
        body,
        out_type=jax.ShapeDtypeStruct((_NC * N, H), jnp.float32),
        mesh=mesh,
        scratch_types=[
            pltpu.VMEM((NI * 3, K), jnp.int32),
            pltpu.VMEM((NBUF * K, H), jnp.float32),
            pltpu.VMEM_SHARED((N, H), jnp.float32),
            pltpu.SemaphoreType.DMA((NBUF,)),
            pltpu.SemaphoreType.DMA((NBUF,)),
            pltpu.SemaphoreType.DMA((NI,)),
        ],
        compiler_params=pltpu.CompilerParams(needs_layout_passes=False),
    )

    NW = _NC * _NS

    def spmm(x, epack):
        return k_fn(x, epack)

    def pack_edges(src, dst, w):
        wbits = lax.bitcast_convert_type(w, jnp.int32)
        e = jnp.stack([src, dst, wbits])
        e = e.reshape(3, NW, NCH, K)
        return e.transpose(1, 2, 0, 3)

    return spmm, pack_edges


def _relu_linear(x, W, b):
    N, F = x.shape
    H = W.shape[1]
    R = 1000

    def body(x_ref, w_ref, b_ref, o_ref):
        o_ref[...] = jnp.maximum(
            jnp.dot(x_ref[...], w_ref[...],
                    preferred_element_type=jnp.float32) + b_ref[...], 0.0)

    return pl.pallas_call(
        body,
        grid=(N // R,),
        in_specs=[pl.BlockSpec((R, F), lambda i: (i, 0)),
                  pl.BlockSpec((F, H), lambda i: (0, 0)),
                  pl.BlockSpec((1, H), lambda i: (0, 0))],
        out_specs=pl.BlockSpec((R, H), lambda i: (i, 0)),
        out_shape=jax.ShapeDtypeStruct((N, H), jnp.float32),
    )(x, W, b)


def _linear(x, W, b):
    N, H = x.shape
    C = W.shape[1]
    R = 1000

    def body(x_ref, w_ref, b_ref, o_ref):
        o_ref[...] = jnp.dot(x_ref[...], w_ref[...],
                             preferred_element_type=jnp.float32) + b_ref[...]

    return pl.pallas_call(
        body,
        grid=(N // R,),
        in_specs=[pl.BlockSpec((R, H), lambda i: (i, 0)),
                  pl.BlockSpec((H, C), lambda i: (0, 0)),
                  pl.BlockSpec((1, C), lambda i: (0, 0))],
        out_specs=pl.BlockSpec((R, C), lambda i: (i, 0)),
        out_shape=jax.ShapeDtypeStruct((N, C), jnp.float32),
    )(x, W, b)


def _gcn2_layer(agg2, x0, x, Wl, beta):
    N, H = x.shape
    R = 1000
    nb = N // R

    def body(a0_ref, a1_ref, x0_ref, x_ref, w_ref, o_ref):
        h = ((1.0 - ALPHA) * (a0_ref[...] + a1_ref[...])
             + ALPHA * x0_ref[...])
        o_ref[...] = jnp.maximum(
            (1.0 - beta) * h
            + beta * jnp.dot(h, w_ref[...], preferred_element_type=jnp.float32)
            + x_ref[...], 0.0)

    return pl.pallas_call(
        body,
        grid=(nb,),
        in_specs=[pl.BlockSpec((R, H), lambda i: (i, 0)),
                  pl.BlockSpec((R, H), lambda i, _nb=nb: (i + _nb, 0)),
                  pl.BlockSpec((R, H), lambda i: (i, 0)),
                  pl.BlockSpec((R, H), lambda i: (i, 0)),
                  pl.BlockSpec((H, H), lambda i: (0, 0))],
        out_specs=pl.BlockSpec((R, H), lambda i: (i, 0)),
        out_shape=jax.ShapeDtypeStruct((N, H), jnp.float32),
    )(agg2, agg2, x0, x, Wl)


def kernel(x, edge_index, edge_weight, W_in, b_in, convW, W_out, b_out):
    N = x.shape[0]
    H = W_in.shape[1]
    E = edge_weight.shape[0]
    L = convW.shape[0]
    src = edge_index[0]
    dst = edge_index[1]
    spmm, pack_edges = _make_spmm(N, E, H)
    epack = pack_edges(src, dst, edge_weight)

    h = _relu_linear(x, W_in, b_in.reshape(1, -1))
    x0 = h
    for l in range(L):
        agg2 = spmm(h, epack)
        beta = float(np.log(THETA / (l + 1) + 1.0))
        h = _gcn2_layer(agg2, x0, h, convW[l], beta)
    return _linear(h, W_out, b_out.reshape(1, -1))
